# baseline (speedup 1.0000x reference)
import functools

import jax
import jax.numpy as jnp
from jax import lax
from jax.experimental import pallas as pl
from jax.experimental.pallas import tpu as pltpu
from jax.experimental.pallas import tpu_sc as plsc

N = 10000
D = 128
E = 320000
NC = 2
NS = 16
LANES = 128
CHUNKS = 80
EW = CHUNKS * LANES
EPAD = NC * NS * EW
RPS = 624
TAIL = N - NS * RPS
HIST = 10240
HRPS = HIST // NS

_mesh = plsc.VectorSubcoreMesh(core_axis_name="c", subcore_axis_name="s")


def _deg_partials(dst3):
    @functools.partial(
        pl.kernel,
        out_type=jax.ShapeDtypeStruct((NC, HIST), jnp.float32),
        mesh=_mesh,
        scratch_types=[
            pltpu.VMEM((CHUNKS, LANES), jnp.int32),
            pltpu.VMEM((HRPS,), jnp.float32),
            pltpu.VMEM((LANES,), jnp.float32),
            pltpu.VMEM_SHARED((HIST,), jnp.float32),
            pltpu.SemaphoreType.DMA,
        ],
    )
    def k(dst_hbm, out_hbm, dst_loc, zbuf, ones, hist_sh, dsem):
        cid = lax.axis_index("c")
        sid = lax.axis_index("s")
        wid = cid * NS + sid
        pltpu.async_copy(dst_hbm.at[wid], dst_loc, dsem)

        def fillz(i, _):
            zbuf[pl.ds(i * 16, 16)] = jnp.zeros((16,), jnp.float32)
            return 0

        lax.fori_loop(0, HRPS // 16, fillz, 0)

        def fillo(i, _):
            ones[pl.ds(i * 16, 16)] = jnp.ones((16,), jnp.float32)
            return 0

        lax.fori_loop(0, LANES // 16, fillo, 0)

        pltpu.sync_copy(zbuf, hist_sh.at[pl.ds(sid * HRPS, HRPS)])
        pltpu.make_async_copy(dst_hbm.at[wid], dst_loc, dsem).wait()
        plsc.subcore_barrier()

        def body(j, _):
            pltpu.sync_copy(ones, hist_sh.at[dst_loc.at[j]], add=True)
            return 0

        lax.fori_loop(0, CHUNKS, body, 0)
        plsc.subcore_barrier()
        pltpu.sync_copy(
            hist_sh.at[pl.ds(sid * HRPS, HRPS)],
            out_hbm.at[cid, pl.ds(sid * HRPS, HRPS)],
        )

    return k(dst3)


def _scatter_partials(z, si3):
    @functools.partial(
        pl.kernel,
        out_type=jax.ShapeDtypeStruct((NC, N, D), jnp.float32),
        mesh=_mesh,
        scratch_types=[
            pltpu.VMEM((4, 2, LANES), jnp.int32),
            pltpu.VMEM((2, LANES, D), jnp.float32),
            pltpu.VMEM_SHARED((N + 16, D), jnp.float32),
            pltpu.SemaphoreType.DMA((2,)),
            pltpu.SemaphoreType.DMA((4,)),
            pltpu.SemaphoreType.DMA((2,)),
        ],
    )
    def k(z_hbm, si_hbm, out_hbm, ibuf, rows, acc_sh, gsem, isem, ssem):
        cid = lax.axis_index("c")
        sid = lax.axis_index("s")
        wid = cid * NS + sid
        pltpu.sync_copy(si_hbm.at[wid, 0], ibuf.at[0])
        pltpu.async_copy(si_hbm.at[wid, 1], ibuf.at[1], isem.at[1])
        pltpu.async_copy(z_hbm.at[ibuf.at[0, 0]], rows.at[0], gsem.at[0])

        pltpu.sync_copy(
            z_hbm.at[pl.ds(sid * RPS, RPS)],
            acc_sh.at[pl.ds(sid * RPS, RPS)],
        )

        @pl.when(sid == 0)
        def _():
            pltpu.sync_copy(
                z_hbm.at[pl.ds(NS * RPS, TAIL)],
                acc_sh.at[pl.ds(NS * RPS, TAIL)],
            )

        plsc.subcore_barrier()


        def body(i, _):
            for q in (0, 1, 2, 3):
                j = 4 * i + q
                b = q % 2

                @pl.when(j >= 1)
                def _():
                    pltpu.make_async_copy(
                        rows.at[1 - b],
                        acc_sh.at[ibuf.at[(q - 1) % 4, 1]],
                        ssem.at[1 - b],
                    ).wait()

                @pl.when(j + 1 < CHUNKS)
                def _():
                    pltpu.make_async_copy(
                        si_hbm.at[wid, j + 1],
                        ibuf.at[(q + 1) % 4],
                        isem.at[(q + 1) % 4],
                    ).wait()
                    pltpu.async_copy(
                        z_hbm.at[ibuf.at[(q + 1) % 4, 0]],
                        rows.at[1 - b],
                        gsem.at[1 - b],
                    )

                pltpu.make_async_copy(
                    z_hbm.at[ibuf.at[q, 0]], rows.at[b], gsem.at[b]
                ).wait()
                pltpu.async_copy(
                    rows.at[b], acc_sh.at[ibuf.at[q, 1]], ssem.at[b], add=True
                )

                @pl.when(j + 2 < CHUNKS)
                def _():
                    pltpu.async_copy(
                        si_hbm.at[wid, j + 2],
                        ibuf.at[(q + 2) % 4],
                        isem.at[(q + 2) % 4],
                    )
            return 0

        lax.fori_loop(0, CHUNKS // 4, body, 0)
        pltpu.make_async_copy(
            rows.at[1], acc_sh.at[ibuf.at[(CHUNKS - 1) % 4, 1]], ssem.at[1]
        ).wait()
        plsc.subcore_barrier()
        pltpu.sync_copy(
            acc_sh.at[pl.ds(sid * RPS, RPS)],
            out_hbm.at[cid, pl.ds(sid * RPS, RPS)],
        )

        @pl.when(sid == 0)
        def _():
            pltpu.sync_copy(
                acc_sh.at[pl.ds(NS * RPS, TAIL)],
                out_hbm.at[cid, pl.ds(NS * RPS, TAIL)],
            )

    return k(z, si3)


_BN = 2000


def _tc_fc(x, W_fc, b_fc, W1, degT):
    def body(x_ref, wfc_ref, bfc_ref, w1_ref, degT_ref, z1_ref, dinv_ref):
        deg = degT_ref[:, 0:1] + degT_ref[:, 1:2] + 1.0
        dinv = lax.rsqrt(deg)
        h = jnp.dot(x_ref[...], wfc_ref[...], preferred_element_type=jnp.float32)
        h = jnp.maximum(h + bfc_ref[...], 0.0)
        y1 = jnp.dot(h, w1_ref[...], preferred_element_type=jnp.float32)
        z1_ref[...] = y1 * dinv
        dinv_ref[...] = dinv

    return pl.pallas_call(
        body,
        grid=(N // _BN,),
        in_specs=[
            pl.BlockSpec((_BN, D), lambda i: (i, 0)),
            pl.BlockSpec((D, D), lambda i: (0, 0)),
            pl.BlockSpec((1, D), lambda i: (0, 0)),
            pl.BlockSpec((D, D), lambda i: (0, 0)),
            pl.BlockSpec((_BN, 2), lambda i: (i, 0)),
        ],
        out_specs=[
            pl.BlockSpec((_BN, D), lambda i: (i, 0)),
            pl.BlockSpec((_BN, 1), lambda i: (i, 0)),
        ],
        out_shape=[
            jax.ShapeDtypeStruct((N, D), jnp.float32),
            jax.ShapeDtypeStruct((N, 1), jnp.float32),
        ],
    )(x, W_fc, b_fc, W1, degT)


def _tc_mid(s1, z1, dinv, b1, W2):
    def body(s_ref, z_ref, dinv_ref, b_ref, w2_ref, z2_ref):
        agg = dinv_ref[...] * (s_ref[0] + s_ref[1] - z_ref[...])
        h2 = jnp.maximum(agg + b_ref[...], 0.0)
        y2 = jnp.dot(h2, w2_ref[...], preferred_element_type=jnp.float32)
        z2_ref[...] = y2 * dinv_ref[...]

    return pl.pallas_call(
        body,
        grid=(N // _BN,),
        in_specs=[
            pl.BlockSpec((NC, _BN, D), lambda i: (0, i, 0)),
            pl.BlockSpec((_BN, D), lambda i: (i, 0)),
            pl.BlockSpec((_BN, 1), lambda i: (i, 0)),
            pl.BlockSpec((1, D), lambda i: (0, 0)),
            pl.BlockSpec((D, D), lambda i: (0, 0)),
        ],
        out_specs=pl.BlockSpec((_BN, D), lambda i: (i, 0)),
        out_shape=jax.ShapeDtypeStruct((N, D), jnp.float32),
    )(s1, z1, dinv, b1, W2)


def _tc_out(s2, z2, dinv, b2):
    def body(s_ref, z_ref, dinv_ref, b_ref, o_ref):
        o_ref[...] = dinv_ref[...] * (s_ref[0] + s_ref[1] - z_ref[...]) + b_ref[...]

    return pl.pallas_call(
        body,
        grid=(N // _BN,),
        in_specs=[
            pl.BlockSpec((NC, _BN, D), lambda i: (0, i, 0)),
            pl.BlockSpec((_BN, D), lambda i: (i, 0)),
            pl.BlockSpec((_BN, 1), lambda i: (i, 0)),
            pl.BlockSpec((1, D), lambda i: (0, 0)),
        ],
        out_specs=pl.BlockSpec((_BN, D), lambda i: (i, 0)),
        out_shape=jax.ShapeDtypeStruct((N, D), jnp.float32),
    )(s2, z2, dinv, b2)


def kernel(x, edge_index, W_fc, b_fc, W1, b1, W2, b2):
    src = edge_index[0].astype(jnp.int32)
    dst = edge_index[1].astype(jnp.int32)
    nw = NC * NS
    ppw = (EPAD - E) // nw
    dummy = (jnp.arange(ppw, dtype=jnp.int32) % 16)[None, :].repeat(nw, axis=0)
    src3 = jnp.concatenate(
        [src.reshape(nw, E // nw), dummy], axis=1).reshape(nw, CHUNKS, LANES)
    dst3 = jnp.concatenate(
        [dst.reshape(nw, E // nw), dummy + N], axis=1).reshape(nw, CHUNKS, LANES)
    si3 = jnp.stack([src3, dst3], axis=2)

    degp = _deg_partials(dst3)
    degT = jnp.transpose(degp)[:N]

    b_fc2 = b_fc.reshape(1, D)
    b12 = b1.reshape(1, D)
    b22 = b2.reshape(1, D)

    z1, dinv = _tc_fc(x, W_fc, b_fc2, W1, degT)
    s1 = _scatter_partials(z1, si3)
    z2 = _tc_mid(s1, z1, dinv, b12, W2)
    s2 = _scatter_partials(z2, si3)
    return _tc_out(s2, z2, dinv, b22)


# speedup vs baseline: 1.0791x; 1.0791x over previous
import functools

import jax
import jax.numpy as jnp
from jax import lax
from jax.experimental import pallas as pl
from jax.experimental.pallas import tpu as pltpu
from jax.experimental.pallas import tpu_sc as plsc

N = 10000
D = 128
E = 320000
NC = 2
NS = 16
LANES = 128
CHUNKS = 79
EW = CHUNKS * LANES
EPAD = NC * NS * EW
RPS = 624
TAIL = N - NS * RPS
HIST = 10240
HRPS = HIST // NS

_mesh = plsc.VectorSubcoreMesh(core_axis_name="c", subcore_axis_name="s")


def _deg_partials(dst3):
    @functools.partial(
        pl.kernel,
        out_type=jax.ShapeDtypeStruct((NC, HIST), jnp.float32),
        mesh=_mesh,
        scratch_types=[
            pltpu.VMEM((CHUNKS, LANES), jnp.int32),
            pltpu.VMEM((HRPS,), jnp.float32),
            pltpu.VMEM((LANES,), jnp.float32),
            pltpu.VMEM_SHARED((HIST,), jnp.float32),
            pltpu.SemaphoreType.DMA,
            pltpu.SemaphoreType.DMA,
        ],
    )
    def k(dst_hbm, out_hbm, dst_loc, zbuf, ones, hist_sh, dsem, hsem):
        cid = lax.axis_index("c")
        sid = lax.axis_index("s")
        wid = cid * NS + sid
        pltpu.async_copy(dst_hbm.at[wid], dst_loc, dsem)

        def fillz(i, _):
            zbuf[pl.ds(i * 16, 16)] = jnp.zeros((16,), jnp.float32)
            return 0

        lax.fori_loop(0, HRPS // 16, fillz, 0)

        def fillo(i, _):
            ones[pl.ds(i * 16, 16)] = jnp.ones((16,), jnp.float32)
            return 0

        lax.fori_loop(0, LANES // 16, fillo, 0)

        pltpu.sync_copy(zbuf, hist_sh.at[pl.ds(sid * HRPS, HRPS)])
        pltpu.make_async_copy(dst_hbm.at[wid], dst_loc, dsem).wait()
        plsc.subcore_barrier()

        def fire(j, _):
            pltpu.async_copy(ones, hist_sh.at[dst_loc.at[j]], hsem, add=True)
            return 0

        def drain(j, _):
            pltpu.make_async_copy(
                ones, hist_sh.at[dst_loc.at[j]], hsem
            ).wait()
            return 0

        lax.fori_loop(0, 4, fire, 0)

        def body(j, _):
            drain(j - 4, 0)
            fire(j, 0)
            return 0

        lax.fori_loop(4, CHUNKS, body, 0)
        lax.fori_loop(CHUNKS - 4, CHUNKS, drain, 0)
        plsc.subcore_barrier()
        pltpu.sync_copy(
            hist_sh.at[pl.ds(sid * HRPS, HRPS)],
            out_hbm.at[cid, pl.ds(sid * HRPS, HRPS)],
        )

    return k(dst3)


def _scatter_partials(z, si3):
    @functools.partial(
        pl.kernel,
        out_type=jax.ShapeDtypeStruct((NC, N, D), jnp.float32),
        mesh=_mesh,
        scratch_types=[
            pltpu.VMEM((4, 2, LANES), jnp.int32),
            pltpu.VMEM((2, LANES, D), jnp.float32),
            pltpu.VMEM_SHARED((N + 16, D), jnp.float32),
            pltpu.SemaphoreType.DMA((2,)),
            pltpu.SemaphoreType.DMA((4,)),
            pltpu.SemaphoreType.DMA((2,)),
        ],
    )
    def k(z_hbm, si_hbm, out_hbm, ibuf, rows, acc_sh, gsem, isem, ssem):
        cid = lax.axis_index("c")
        sid = lax.axis_index("s")
        wid = cid * NS + sid
        pltpu.sync_copy(si_hbm.at[wid, 0], ibuf.at[0])
        pltpu.async_copy(si_hbm.at[wid, 1], ibuf.at[1], isem.at[1])
        pltpu.async_copy(z_hbm.at[ibuf.at[0, 0]], rows.at[0], gsem.at[0])

        pltpu.sync_copy(
            z_hbm.at[pl.ds(sid * RPS, RPS)],
            acc_sh.at[pl.ds(sid * RPS, RPS)],
        )

        @pl.when(sid == 0)
        def _():
            pltpu.sync_copy(
                z_hbm.at[pl.ds(NS * RPS, TAIL)],
                acc_sh.at[pl.ds(NS * RPS, TAIL)],
            )

        plsc.subcore_barrier()


        def step(j, q, b, first, do_gather, do_idx):
            if not first:
                pltpu.make_async_copy(
                    rows.at[1 - b],
                    acc_sh.at[ibuf.at[(q - 1) % 4, 1]],
                    ssem.at[1 - b],
                ).wait()
            if do_gather:
                pltpu.make_async_copy(
                    si_hbm.at[wid, j + 1],
                    ibuf.at[(q + 1) % 4],
                    isem.at[(q + 1) % 4],
                ).wait()
                pltpu.async_copy(
                    z_hbm.at[ibuf.at[(q + 1) % 4, 0]],
                    rows.at[1 - b],
                    gsem.at[1 - b],
                )
            pltpu.make_async_copy(
                z_hbm.at[ibuf.at[q, 0]], rows.at[b], gsem.at[b]
            ).wait()
            pltpu.async_copy(
                rows.at[b], acc_sh.at[ibuf.at[q, 1]], ssem.at[b], add=True
            )
            if do_idx:
                pltpu.async_copy(
                    si_hbm.at[wid, j + 2],
                    ibuf.at[(q + 2) % 4],
                    isem.at[(q + 2) % 4],
                )

        def first_body(i, _):
            for q in (0, 1, 2, 3):
                step(4 * i + q, q, q % 2, i == 0 and q == 0, True, True)
            return 0

        def body(i, _):
            for q in (0, 1, 2, 3):
                step(4 * i + q, q, q % 2, False, True, True)
            return 0

        first_body(0, 0)
        lax.fori_loop(1, 19, body, 0)
        step(76, 0, 0, False, True, True)
        step(77, 1, 1, False, True, False)
        step(78, 2, 0, False, False, False)
        pltpu.make_async_copy(
            rows.at[0], acc_sh.at[ibuf.at[2, 1]], ssem.at[0]
        ).wait()
        plsc.subcore_barrier()
        pltpu.sync_copy(
            acc_sh.at[pl.ds(sid * RPS, RPS)],
            out_hbm.at[cid, pl.ds(sid * RPS, RPS)],
        )

        @pl.when(sid == 0)
        def _():
            pltpu.sync_copy(
                acc_sh.at[pl.ds(NS * RPS, TAIL)],
                out_hbm.at[cid, pl.ds(NS * RPS, TAIL)],
            )

    return k(z, si3)


_BN = 2000


def _tc_fc(x, W_fc, b_fc, W1, degT):
    def body(x_ref, wfc_ref, bfc_ref, w1_ref, degT_ref, z1_ref, dinv_ref):
        deg = degT_ref[:, 0:1] + degT_ref[:, 1:2] + 1.0
        dinv = lax.rsqrt(deg)
        h = jnp.dot(x_ref[...], wfc_ref[...], preferred_element_type=jnp.float32)
        h = jnp.maximum(h + bfc_ref[...], 0.0)
        y1 = jnp.dot(h, w1_ref[...], preferred_element_type=jnp.float32)
        z1_ref[...] = y1 * dinv
        dinv_ref[...] = dinv

    return pl.pallas_call(
        body,
        grid=(N // _BN,),
        in_specs=[
            pl.BlockSpec((_BN, D), lambda i: (i, 0)),
            pl.BlockSpec((D, D), lambda i: (0, 0)),
            pl.BlockSpec((1, D), lambda i: (0, 0)),
            pl.BlockSpec((D, D), lambda i: (0, 0)),
            pl.BlockSpec((_BN, 2), lambda i: (i, 0)),
        ],
        out_specs=[
            pl.BlockSpec((_BN, D), lambda i: (i, 0)),
            pl.BlockSpec((_BN, 1), lambda i: (i, 0)),
        ],
        out_shape=[
            jax.ShapeDtypeStruct((N, D), jnp.float32),
            jax.ShapeDtypeStruct((N, 1), jnp.float32),
        ],
    )(x, W_fc, b_fc, W1, degT)


def _tc_mid(s1, z1, dinv, b1, W2):
    def body(s_ref, z_ref, dinv_ref, b_ref, w2_ref, z2_ref):
        agg = dinv_ref[...] * (s_ref[0] + s_ref[1] - z_ref[...])
        h2 = jnp.maximum(agg + b_ref[...], 0.0)
        y2 = jnp.dot(h2, w2_ref[...], preferred_element_type=jnp.float32)
        z2_ref[...] = y2 * dinv_ref[...]

    return pl.pallas_call(
        body,
        grid=(N // _BN,),
        in_specs=[
            pl.BlockSpec((NC, _BN, D), lambda i: (0, i, 0)),
            pl.BlockSpec((_BN, D), lambda i: (i, 0)),
            pl.BlockSpec((_BN, 1), lambda i: (i, 0)),
            pl.BlockSpec((1, D), lambda i: (0, 0)),
            pl.BlockSpec((D, D), lambda i: (0, 0)),
        ],
        out_specs=pl.BlockSpec((_BN, D), lambda i: (i, 0)),
        out_shape=jax.ShapeDtypeStruct((N, D), jnp.float32),
    )(s1, z1, dinv, b1, W2)


def _tc_out(s2, z2, dinv, b2):
    def body(s_ref, z_ref, dinv_ref, b_ref, o_ref):
        o_ref[...] = dinv_ref[...] * (s_ref[0] + s_ref[1] - z_ref[...]) + b_ref[...]

    return pl.pallas_call(
        body,
        grid=(N // _BN,),
        in_specs=[
            pl.BlockSpec((NC, _BN, D), lambda i: (0, i, 0)),
            pl.BlockSpec((_BN, D), lambda i: (i, 0)),
            pl.BlockSpec((_BN, 1), lambda i: (i, 0)),
            pl.BlockSpec((1, D), lambda i: (0, 0)),
        ],
        out_specs=pl.BlockSpec((_BN, D), lambda i: (i, 0)),
        out_shape=jax.ShapeDtypeStruct((N, D), jnp.float32),
    )(s2, z2, dinv, b2)


def kernel(x, edge_index, W_fc, b_fc, W1, b1, W2, b2):
    src = edge_index[0].astype(jnp.int32)
    dst = edge_index[1].astype(jnp.int32)
    nw = NC * NS
    ppw = (EPAD - E) // nw
    dummy = (jnp.arange(ppw, dtype=jnp.int32) % 16)[None, :].repeat(nw, axis=0)
    src3 = jnp.concatenate(
        [src.reshape(nw, E // nw), dummy], axis=1).reshape(nw, CHUNKS, LANES)
    dst3 = jnp.concatenate(
        [dst.reshape(nw, E // nw), dummy + N], axis=1).reshape(nw, CHUNKS, LANES)
    si3 = jnp.stack([src3, dst3], axis=2)

    degp = _deg_partials(dst3)
    degT = jnp.transpose(degp)[:N]

    b_fc2 = b_fc.reshape(1, D)
    b12 = b1.reshape(1, D)
    b22 = b2.reshape(1, D)

    z1, dinv = _tc_fc(x, W_fc, b_fc2, W1, degT)
    s1 = _scatter_partials(z1, si3)
    z2 = _tc_mid(s1, z1, dinv, b12, W2)
    s2 = _scatter_partials(z2, si3)
    return _tc_out(s2, z2, dinv, b22)


# speedup vs baseline: 1.0832x; 1.0038x over previous
import functools

import jax
import jax.numpy as jnp
from jax import lax
from jax.experimental import pallas as pl
from jax.experimental.pallas import tpu as pltpu
from jax.experimental.pallas import tpu_sc as plsc

N = 10000
D = 128
E = 320000
NC = 2
NS = 16
LANES = 128
CHUNKS = 79
CDATA = 80
EW = CDATA * LANES
EPAD = NC * NS * EW
RPS = 624
TAIL = N - NS * RPS
HIST = 10240
HRPS = HIST // NS

_mesh = plsc.VectorSubcoreMesh(core_axis_name="c", subcore_axis_name="s")


def _deg_partials(dst3):
    @functools.partial(
        pl.kernel,
        out_type=jax.ShapeDtypeStruct((NC, HIST), jnp.float32),
        mesh=_mesh,
        scratch_types=[
            pltpu.VMEM((CDATA, LANES), jnp.int32),
            pltpu.VMEM((HRPS,), jnp.float32),
            pltpu.VMEM((LANES,), jnp.float32),
            pltpu.VMEM_SHARED((HIST,), jnp.float32),
            pltpu.SemaphoreType.DMA,
            pltpu.SemaphoreType.DMA,
        ],
    )
    def k(dst_hbm, out_hbm, dst_loc, zbuf, ones, hist_sh, dsem, hsem):
        cid = lax.axis_index("c")
        sid = lax.axis_index("s")
        wid = cid * NS + sid
        pltpu.async_copy(dst_hbm.at[wid], dst_loc, dsem)

        def fillz(i, _):
            zbuf[pl.ds(i * 16, 16)] = jnp.zeros((16,), jnp.float32)
            return 0

        lax.fori_loop(0, HRPS // 16, fillz, 0)

        def fillo(i, _):
            ones[pl.ds(i * 16, 16)] = jnp.ones((16,), jnp.float32)
            return 0

        lax.fori_loop(0, LANES // 16, fillo, 0)

        pltpu.sync_copy(zbuf, hist_sh.at[pl.ds(sid * HRPS, HRPS)])
        pltpu.make_async_copy(dst_hbm.at[wid], dst_loc, dsem).wait()
        plsc.subcore_barrier()

        def fire(j, _):
            pltpu.async_copy(ones, hist_sh.at[dst_loc.at[j]], hsem, add=True)
            return 0

        def drain(j, _):
            pltpu.make_async_copy(
                ones, hist_sh.at[dst_loc.at[j]], hsem
            ).wait()
            return 0

        lax.fori_loop(0, 4, fire, 0)

        def body(j, _):
            drain(j - 4, 0)
            fire(j, 0)
            return 0

        lax.fori_loop(4, CHUNKS, body, 0)
        lax.fori_loop(CHUNKS - 4, CHUNKS, drain, 0)
        plsc.subcore_barrier()
        pltpu.sync_copy(
            hist_sh.at[pl.ds(sid * HRPS, HRPS)],
            out_hbm.at[cid, pl.ds(sid * HRPS, HRPS)],
        )

    return k(dst3)


def _scatter_partials(z, si3):
    @functools.partial(
        pl.kernel,
        out_type=jax.ShapeDtypeStruct((NC, N, D), jnp.float32),
        mesh=_mesh,
        scratch_types=[
            pltpu.VMEM((2, 4, 2, LANES), jnp.int32),
            pltpu.VMEM((2, LANES, D), jnp.float32),
            pltpu.VMEM_SHARED((N + 16, D), jnp.float32),
            pltpu.SemaphoreType.DMA((2,)),
            pltpu.SemaphoreType.DMA((2,)),
            pltpu.SemaphoreType.DMA((2,)),
        ],
    )
    def k(z_hbm, si_hbm, out_hbm, ibuf, rows, acc_sh, gsem, isem, ssem):
        cid = lax.axis_index("c")
        sid = lax.axis_index("s")
        wid = cid * NS + sid
        pltpu.sync_copy(si_hbm.at[wid, pl.ds(0, 4)], ibuf.at[0])
        pltpu.async_copy(si_hbm.at[wid, pl.ds(4, 4)], ibuf.at[1], isem.at[1])
        pltpu.async_copy(z_hbm.at[ibuf.at[0, 0, 0]], rows.at[0], gsem.at[0])

        pltpu.sync_copy(
            z_hbm.at[pl.ds(sid * RPS, RPS)],
            acc_sh.at[pl.ds(sid * RPS, RPS)],
        )

        @pl.when(sid == 0)
        def _():
            pltpu.sync_copy(
                z_hbm.at[pl.ds(NS * RPS, TAIL)],
                acc_sh.at[pl.ds(NS * RPS, TAIL)],
            )

        plsc.subcore_barrier()


        def step(q, b, B, first, loadg, gsrc):
            if not first:
                pltpu.make_async_copy(
                    rows.at[1 - b],
                    acc_sh.at[ibuf.at[B, q, 1]],
                    ssem.at[1 - b],
                ).wait()
            if loadg is not None:
                pltpu.async_copy(
                    si_hbm.at[wid, pl.ds(4 * loadg, 4)],
                    ibuf.at[1 - B],
                    isem.at[1 - B],
                )
            if gsrc is not None:
                gb, gp = gsrc
                if gb != B:
                    pltpu.make_async_copy(
                        si_hbm.at[wid, pl.ds(0, 4)], ibuf.at[gb], isem.at[gb]
                    ).wait()
                pltpu.async_copy(
                    z_hbm.at[ibuf.at[gb, gp, 0]],
                    rows.at[1 - b],
                    gsem.at[1 - b],
                )
            pltpu.make_async_copy(
                z_hbm.at[ibuf.at[B, q, 0]], rows.at[b], gsem.at[b]
            ).wait()
            pltpu.async_copy(
                rows.at[b], acc_sh.at[ibuf.at[B, q, 1]], ssem.at[b], add=True
            )

        for q in (0, 1, 2, 3):
            step(q, q % 2, 0, q == 0, None,
                 (0, q + 1) if q < 3 else (1, 0))

        def body(i, _):
            for G in (1, 2):
                g = 2 * i + G
                B = G % 2
                for q in (0, 1, 2, 3):
                    step(q, q % 2, B,
                         False,
                         g + 1 if q == 0 else None,
                         (B, q + 1) if q < 3 else (1 - B, 0))
            return 0

        lax.fori_loop(0, 9, body, 0)
        step(0, 0, 1, False, None, (1, 1))
        step(1, 1, 1, False, None, (1, 2))
        step(2, 0, 1, False, None, None)
        pltpu.make_async_copy(
            rows.at[0], acc_sh.at[ibuf.at[1, 2, 1]], ssem.at[0]
        ).wait()
        plsc.subcore_barrier()
        pltpu.sync_copy(
            acc_sh.at[pl.ds(sid * RPS, RPS)],
            out_hbm.at[cid, pl.ds(sid * RPS, RPS)],
        )

        @pl.when(sid == 0)
        def _():
            pltpu.sync_copy(
                acc_sh.at[pl.ds(NS * RPS, TAIL)],
                out_hbm.at[cid, pl.ds(NS * RPS, TAIL)],
            )

    return k(z, si3)


_BN = 2000


def _tc_fc(x, W_fc, b_fc, W1, degT):
    def body(x_ref, wfc_ref, bfc_ref, w1_ref, degT_ref, z1_ref, dinv_ref):
        deg = degT_ref[:, 0:1] + degT_ref[:, 1:2] + 1.0
        dinv = lax.rsqrt(deg)
        h = jnp.dot(x_ref[...], wfc_ref[...], preferred_element_type=jnp.float32)
        h = jnp.maximum(h + bfc_ref[...], 0.0)
        y1 = jnp.dot(h, w1_ref[...], preferred_element_type=jnp.float32)
        z1_ref[...] = y1 * dinv
        dinv_ref[...] = dinv

    return pl.pallas_call(
        body,
        grid=(N // _BN,),
        in_specs=[
            pl.BlockSpec((_BN, D), lambda i: (i, 0)),
            pl.BlockSpec((D, D), lambda i: (0, 0)),
            pl.BlockSpec((1, D), lambda i: (0, 0)),
            pl.BlockSpec((D, D), lambda i: (0, 0)),
            pl.BlockSpec((_BN, 2), lambda i: (i, 0)),
        ],
        out_specs=[
            pl.BlockSpec((_BN, D), lambda i: (i, 0)),
            pl.BlockSpec((_BN, 1), lambda i: (i, 0)),
        ],
        out_shape=[
            jax.ShapeDtypeStruct((N, D), jnp.float32),
            jax.ShapeDtypeStruct((N, 1), jnp.float32),
        ],
    )(x, W_fc, b_fc, W1, degT)


def _tc_mid(s1, z1, dinv, b1, W2):
    def body(s_ref, z_ref, dinv_ref, b_ref, w2_ref, z2_ref):
        agg = dinv_ref[...] * (s_ref[0] + s_ref[1] - z_ref[...])
        h2 = jnp.maximum(agg + b_ref[...], 0.0)
        y2 = jnp.dot(h2, w2_ref[...], preferred_element_type=jnp.float32)
        z2_ref[...] = y2 * dinv_ref[...]

    return pl.pallas_call(
        body,
        grid=(N // _BN,),
        in_specs=[
            pl.BlockSpec((NC, _BN, D), lambda i: (0, i, 0)),
            pl.BlockSpec((_BN, D), lambda i: (i, 0)),
            pl.BlockSpec((_BN, 1), lambda i: (i, 0)),
            pl.BlockSpec((1, D), lambda i: (0, 0)),
            pl.BlockSpec((D, D), lambda i: (0, 0)),
        ],
        out_specs=pl.BlockSpec((_BN, D), lambda i: (i, 0)),
        out_shape=jax.ShapeDtypeStruct((N, D), jnp.float32),
    )(s1, z1, dinv, b1, W2)


def _tc_out(s2, z2, dinv, b2):
    def body(s_ref, z_ref, dinv_ref, b_ref, o_ref):
        o_ref[...] = dinv_ref[...] * (s_ref[0] + s_ref[1] - z_ref[...]) + b_ref[...]

    return pl.pallas_call(
        body,
        grid=(N // _BN,),
        in_specs=[
            pl.BlockSpec((NC, _BN, D), lambda i: (0, i, 0)),
            pl.BlockSpec((_BN, D), lambda i: (i, 0)),
            pl.BlockSpec((_BN, 1), lambda i: (i, 0)),
            pl.BlockSpec((1, D), lambda i: (0, 0)),
        ],
        out_specs=pl.BlockSpec((_BN, D), lambda i: (i, 0)),
        out_shape=jax.ShapeDtypeStruct((N, D), jnp.float32),
    )(s2, z2, dinv, b2)


def kernel(x, edge_index, W_fc, b_fc, W1, b1, W2, b2):
    src = edge_index[0].astype(jnp.int32)
    dst = edge_index[1].astype(jnp.int32)
    nw = NC * NS
    ppw = (EPAD - E) // nw
    dummy = (jnp.arange(ppw, dtype=jnp.int32) % 16)[None, :].repeat(nw, axis=0)
    src3 = jnp.concatenate(
        [src.reshape(nw, E // nw), dummy], axis=1).reshape(nw, CDATA, LANES)
    dst3 = jnp.concatenate(
        [dst.reshape(nw, E // nw), dummy + N], axis=1).reshape(nw, CDATA, LANES)
    si3 = jnp.stack([src3, dst3], axis=2)

    degp = _deg_partials(dst3)
    degT = jnp.transpose(degp)[:N]

    b_fc2 = b_fc.reshape(1, D)
    b12 = b1.reshape(1, D)
    b22 = b2.reshape(1, D)

    z1, dinv = _tc_fc(x, W_fc, b_fc2, W1, degT)
    s1 = _scatter_partials(z1, si3)
    z2 = _tc_mid(s1, z1, dinv, b12, W2)
    s2 = _scatter_partials(z2, si3)
    return _tc_out(s2, z2, dinv, b22)
